# Initial kernel scaffold; baseline (speedup 1.0000x reference)
import jax
import jax.numpy as jnp
from jax.experimental import pallas as pl


def kernel(query_embeddings, all_image_embeddings, all_knowledge_embeddings, image_labels, knowledge_labels):
    raise NotImplementedError("write your pallas kernel here")


# speedup vs baseline: 6.4680x; 6.4680x over previous
import functools

import jax
import jax.numpy as jnp
from jax import lax
from jax.experimental import pallas as pl
from jax.experimental.pallas import tpu as pltpu
from jax.experimental.pallas import tpu_sc as plsc

B, N, KPOOL, D, C, R, KR = 64, 50000, 5000, 512, 25, 8, 4
NEG = -1e9
CH = 2048
NPAD = 51200
NCH = NPAD // CH
KPAD = 5120
KWIN = 512
BIGI = 2**30
_SC_CHUNK = 80


def _sc_gather(table, idx):
    Bn = idx.shape[0]
    Dt = table.shape[1]
    info = plsc.get_sparse_core_info()
    NW = info.num_cores * info.num_subcores
    per_w = Bn // NW
    nchunks = per_w // _SC_CHUNK
    mesh = plsc.VectorSubcoreMesh(core_axis_name="c", subcore_axis_name="s")

    @functools.partial(
        pl.kernel,
        mesh=mesh,
        out_type=jax.ShapeDtypeStruct((Bn, Dt), jnp.float32),
        scratch_types=[
            pltpu.VMEM((_SC_CHUNK,), jnp.int32),
            pltpu.VMEM((_SC_CHUNK, Dt), jnp.float32),
            pltpu.SemaphoreType.DMA,
        ],
    )
    def k(table_hbm, idx_hbm, out_hbm, idx_v, rows_v, sem):
        wid = lax.axis_index("s") * info.num_cores + lax.axis_index("c")

        def body(j, _):
            base = wid * per_w + j * _SC_CHUNK
            pltpu.sync_copy(idx_hbm.at[pl.ds(base, _SC_CHUNK)], idx_v)
            pltpu.async_copy(table_hbm.at[idx_v], rows_v, sem).wait()
            pltpu.sync_copy(rows_v, out_hbm.at[pl.ds(base, _SC_CHUNK)])
            return 0

        lax.fori_loop(0, nchunks, body, 0)

    return k(table, idx)


def _stage1_body(q_ref, g_ref, starts_ref, ends_ref, vals_ref, idxs_ref):
    n = pl.program_id(0)

    @pl.when(n == 0)
    def _init():
        vals_ref[...] = jnp.full((C, B, R), NEG, jnp.float32)
        idxs_ref[...] = jnp.zeros((C, B, R), jnp.int32)

    S = lax.dot_general(q_ref[...], g_ref[...], (((1,), (1,)), ((), ())),
                        preferred_element_type=jnp.float32)
    iota = lax.broadcasted_iota(jnp.int32, (B, CH), 1)
    iota16 = lax.broadcasted_iota(jnp.int32, (B, 2 * R), 1)

    def class_body(c, _):
        s = starts_ref[c]
        e = ends_ref[c]

        @pl.when((s < (n + 1) * CH) & (e > n * CH))
        def _active():
            lo = s - n * CH
            hi = e - n * CH
            Sm = jnp.where((iota >= lo) & (iota < hi), S, NEG)
            cand_v = []
            cand_i = []
            cur = Sm
            for _ in range(R):
                m = jnp.max(cur, axis=1, keepdims=True)
                idx = jnp.min(jnp.where(cur == m, iota, BIGI), axis=1, keepdims=True)
                cand_v.append(m)
                cand_i.append(idx + n * CH)
                cur = jnp.where(iota == idx, NEG, cur)
            candv = jnp.concatenate(cand_v, axis=1)
            candi = jnp.concatenate(cand_i, axis=1)
            run_v = vals_ref[pl.ds(c, 1)][0]
            run_i = idxs_ref[pl.ds(c, 1)][0]
            allv = jnp.concatenate([run_v, candv], axis=1)
            alli = jnp.concatenate([run_i, candi], axis=1)
            newv = []
            newi = []
            for _ in range(R):
                m = jnp.max(allv, axis=1, keepdims=True)
                idx = jnp.min(jnp.where(allv == m, iota16, BIGI), axis=1, keepdims=True)
                sel = jnp.sum(jnp.where(iota16 == idx, alli, 0), axis=1, keepdims=True)
                newv.append(m)
                newi.append(sel)
                allv = jnp.where(iota16 == idx, NEG, allv)
            vals_ref[pl.ds(c, 1)] = jnp.concatenate(newv, axis=1)[None]
            idxs_ref[pl.ds(c, 1)] = jnp.concatenate(newi, axis=1)[None]
        return 0

    lax.fori_loop(0, C, class_body, 0)


def _stage1(q, g_sorted, starts, ends):
    return pl.pallas_call(
        _stage1_body,
        grid=(NCH,),
        in_specs=[
            pl.BlockSpec((B, D), lambda n: (0, 0)),
            pl.BlockSpec((CH, D), lambda n: (n, 0)),
            pl.BlockSpec(memory_space=pltpu.MemorySpace.SMEM),
            pl.BlockSpec(memory_space=pltpu.MemorySpace.SMEM),
        ],
        out_specs=[
            pl.BlockSpec((C, B, R), lambda n: (0, 0, 0)),
            pl.BlockSpec((C, B, R), lambda n: (0, 0, 0)),
        ],
        out_shape=[
            jax.ShapeDtypeStruct((C, B, R), jnp.float32),
            jax.ShapeDtypeStruct((C, B, R), jnp.int32),
        ],
    )(q, g_sorted, starts, ends)


def _stage2_body(q_ref, x_ref, w_ref, kstarts_ref, kends_ref, khbm_ref,
                 out_img_ref, out_know_ref, kseg_ref, sem):
    c = pl.program_id(0)
    ks = kstarts_ref[c]
    ke = kends_ref[c]
    base = jnp.minimum((ks // 8) * 8, KPAD - KWIN)
    cp = pltpu.make_async_copy(khbm_ref.at[pl.ds(base, KWIN)], kseg_ref, sem)
    cp.start()
    cp.wait()
    X = x_ref[0]
    Kseg = kseg_ref[...]
    S2 = lax.dot_general(X, Kseg, (((1,), (1,)), ((), ())),
                         preferred_element_type=jnp.float32)
    iota = lax.broadcasted_iota(jnp.int32, (B * R, KWIN), 1)
    colk = iota + base
    Sm = jnp.where((colk >= ks) & (colk < ke), S2, NEG)
    cur = Sm
    m1 = None
    m = None
    for r in range(KR):
        m = jnp.max(cur, axis=1, keepdims=True)
        if r == 0:
            m1 = m
        idx = jnp.min(jnp.where(cur == m, iota, BIGI), axis=1, keepdims=True)
        cur = jnp.where(iota == idx, NEG, cur)
    t_last = m
    Wk = jnp.where(Sm >= t_last, jnp.exp(Sm - m1), 0.0)
    denom = jnp.sum(Wk, axis=1, keepdims=True)
    A = Wk / denom
    per_img = lax.dot_general(A, Kseg, (((1,), (0,)), ((), ())),
                              preferred_element_type=jnp.float32)
    w8 = w_ref[0]
    mw = jnp.max(w8, axis=1, keepdims=True)
    ew = jnp.exp(w8 - mw)
    att = ew / jnp.sum(ew, axis=1, keepdims=True)
    per3 = per_img.reshape(B, R, D)
    X3 = X.reshape(B, R, D)
    ctx_know = jnp.zeros((B, D), jnp.float32)
    ctx_img = jnp.zeros((B, D), jnp.float32)
    for r in range(R):
        a = att[:, r:r + 1]
        ctx_know = ctx_know + a * per3[:, r, :]
        ctx_img = ctx_img + a * X3[:, r, :]
    q = q_ref[...]
    out_img_ref[...] = (0.5 * q + 0.5 * ctx_img)[None]
    out_know_ref[...] = (0.5 * q + 0.5 * ctx_know)[None]


def _stage2(q, x_img, w_img, kstarts, kends, k_sorted):
    return pl.pallas_call(
        _stage2_body,
        grid=(C,),
        in_specs=[
            pl.BlockSpec((B, D), lambda c: (0, 0)),
            pl.BlockSpec((1, B * R, D), lambda c: (c, 0, 0)),
            pl.BlockSpec((1, B, R), lambda c: (c, 0, 0)),
            pl.BlockSpec(memory_space=pltpu.MemorySpace.SMEM),
            pl.BlockSpec(memory_space=pltpu.MemorySpace.SMEM),
            pl.BlockSpec(memory_space=pltpu.MemorySpace.HBM),
        ],
        out_specs=[
            pl.BlockSpec((1, B, D), lambda c: (c, 0, 0)),
            pl.BlockSpec((1, B, D), lambda c: (c, 0, 0)),
        ],
        out_shape=[
            jax.ShapeDtypeStruct((C, B, D), jnp.float32),
            jax.ShapeDtypeStruct((C, B, D), jnp.float32),
        ],
        scratch_shapes=[
            pltpu.VMEM((KWIN, D), jnp.float32),
            pltpu.SemaphoreType.DMA,
        ],
    )(q, x_img, w_img, kstarts, kends, k_sorted)


def kernel(query_embeddings, all_image_embeddings, all_knowledge_embeddings,
           image_labels, knowledge_labels):
    classes = jnp.arange(C, dtype=image_labels.dtype)

    img_order = jnp.argsort(image_labels)
    sl = image_labels[img_order]
    starts = jnp.searchsorted(sl, classes, side='left').astype(jnp.int32)
    ends = jnp.searchsorted(sl, classes, side='right').astype(jnp.int32)
    img_order_p = jnp.concatenate(
        [img_order.astype(jnp.int32), jnp.zeros((NPAD - N,), jnp.int32)])

    korder = jnp.argsort(knowledge_labels)
    kl = knowledge_labels[korder]
    kstarts = jnp.searchsorted(kl, classes, side='left').astype(jnp.int32)
    kends = jnp.searchsorted(kl, classes, side='right').astype(jnp.int32)
    korder_p = jnp.concatenate(
        [korder.astype(jnp.int32), jnp.zeros((KPAD - KPOOL,), jnp.int32)])

    g_sorted = _sc_gather(all_image_embeddings, img_order_p)
    k_sorted = _sc_gather(all_knowledge_embeddings, korder_p)

    vals, idxs = _stage1(query_embeddings, g_sorted, starts, ends)
    x_img = _sc_gather(g_sorted, idxs.reshape(-1)).reshape(C, B * R, D)
    out_img, out_know = _stage2(query_embeddings, x_img, vals, kstarts, kends,
                                k_sorted)
    return (jnp.transpose(out_img, (1, 0, 2)),
            jnp.transpose(out_know, (1, 0, 2)))


# speedup vs baseline: 7.8533x; 1.2142x over previous
import functools

import jax
import jax.numpy as jnp
from jax import lax
from jax.experimental import pallas as pl
from jax.experimental.pallas import tpu as pltpu
from jax.experimental.pallas import tpu_sc as plsc

B, N, KPOOL, D, C, R, KR = 64, 50000, 5000, 512, 25, 8, 4
NEG = -1e9
NPAD = 51200
KPAD = 5120
GW = 2560
KWIN = 384
MCH = 2000
BIGI = 2**30
_SC_CHUNK = 80


def _sc_gather(table, idx):
    Bn = idx.shape[0]
    Dt = table.shape[1]
    info = plsc.get_sparse_core_info()
    NW = info.num_cores * info.num_subcores
    per_w = Bn // NW
    nchunks = per_w // _SC_CHUNK
    mesh = plsc.VectorSubcoreMesh(core_axis_name="c", subcore_axis_name="s")

    @functools.partial(
        pl.kernel,
        mesh=mesh,
        out_type=jax.ShapeDtypeStruct((Bn, Dt), jnp.float32),
        scratch_types=[
            pltpu.VMEM((_SC_CHUNK,), jnp.int32),
            pltpu.VMEM((_SC_CHUNK, Dt), jnp.float32),
            pltpu.SemaphoreType.DMA,
        ],
    )
    def k(table_hbm, idx_hbm, out_hbm, idx_v, rows_v, sem):
        wid = lax.axis_index("s") * info.num_cores + lax.axis_index("c")

        def body(j, _):
            base = wid * per_w + j * _SC_CHUNK
            pltpu.sync_copy(idx_hbm.at[pl.ds(base, _SC_CHUNK)], idx_v)
            pltpu.async_copy(table_hbm.at[idx_v], rows_v, sem).wait()
            pltpu.sync_copy(rows_v, out_hbm.at[pl.ds(base, _SC_CHUNK)])
            return 0

        lax.fori_loop(0, nchunks, body, 0)

    return k(table, idx)


def _t0_body(g_ref, q_ref, out_ref):
    out_ref[...] = lax.dot_general(g_ref[...], q_ref[...], (((1,), (1,)), ((), ())),
                                   preferred_element_type=jnp.float32)


def _t0_sims(g, q_pad):
    return pl.pallas_call(
        _t0_body,
        grid=(N // MCH,),
        in_specs=[
            pl.BlockSpec((MCH, D), lambda n: (n, 0)),
            pl.BlockSpec((2 * B, D), lambda n: (0, 0)),
        ],
        out_specs=pl.BlockSpec((MCH, 2 * B), lambda n: (n, 0)),
        out_shape=jax.ShapeDtypeStruct((N, 2 * B), jnp.float32),
    )(g, q_pad)


def _t1_body(starts_ref, ends_ref, sims_hbm, att_ref, pos_ref, win_ref, sem):
    c = pl.program_id(0)
    s = starts_ref[c]
    e = ends_ref[c]
    base = jnp.minimum((s // 8) * 8, NPAD - GW)
    cp = pltpu.make_async_copy(sims_hbm.at[pl.ds(base, GW)], win_ref, sem)
    cp.start()
    cp.wait()
    S = win_ref[:, :B]
    riota = lax.broadcasted_iota(jnp.int32, (GW, B), 0)
    lo = s - base
    hi = e - base
    S = jnp.where((riota >= lo) & (riota < hi), S, NEG)
    vals = []
    poss = []
    for _ in range(R):
        m = jnp.max(S, axis=0, keepdims=True)
        idx = jnp.min(jnp.where(S == m, riota, BIGI), axis=0, keepdims=True)
        vals.append(m)
        poss.append(idx + base)
        S = jnp.where(riota == idx, NEG, S)
    v8 = jnp.concatenate(vals, axis=0)
    mw = jnp.max(v8, axis=0, keepdims=True)
    ew = jnp.exp(v8 - mw)
    att_ref[...] = (ew / jnp.sum(ew, axis=0, keepdims=True))[None]
    pos_ref[...] = jnp.concatenate(poss, axis=0)[None]


def _t1_topk(starts, ends, sims_sorted):
    return pl.pallas_call(
        _t1_body,
        grid=(C,),
        in_specs=[
            pl.BlockSpec(memory_space=pltpu.MemorySpace.SMEM),
            pl.BlockSpec(memory_space=pltpu.MemorySpace.SMEM),
            pl.BlockSpec(memory_space=pltpu.MemorySpace.HBM),
        ],
        out_specs=[
            pl.BlockSpec((1, R, B), lambda c: (c, 0, 0)),
            pl.BlockSpec((1, R, B), lambda c: (c, 0, 0)),
        ],
        out_shape=[
            jax.ShapeDtypeStruct((C, R, B), jnp.float32),
            jax.ShapeDtypeStruct((C, R, B), jnp.int32),
        ],
        scratch_shapes=[
            pltpu.VMEM((GW, 2 * B), jnp.float32),
            pltpu.SemaphoreType.DMA,
        ],
    )(starts, ends, sims_sorted)


def _t2_body(q_ref, x_ref, att_ref, kstarts_ref, kends_ref, khbm_ref,
             out_img_ref, out_know_ref, kseg_ref, sem):
    c = pl.program_id(0)
    ks = kstarts_ref[c]
    ke = kends_ref[c]
    base = jnp.minimum((ks // 8) * 8, KPAD - KWIN)
    cp = pltpu.make_async_copy(khbm_ref.at[pl.ds(base, KWIN)], kseg_ref, sem)
    cp.start()
    cp.wait()
    X = x_ref[0]
    Kseg = kseg_ref[...]
    S2 = lax.dot_general(X, Kseg, (((1,), (1,)), ((), ())),
                         preferred_element_type=jnp.float32)
    iota = lax.broadcasted_iota(jnp.int32, (R * B, KWIN), 1)
    colk = iota + base
    Sm = jnp.where((colk >= ks) & (colk < ke), S2, NEG)
    cur = Sm
    m1 = None
    m = None
    for r in range(KR):
        m = jnp.max(cur, axis=1, keepdims=True)
        if r == 0:
            m1 = m
        idx = jnp.min(jnp.where(cur == m, iota, BIGI), axis=1, keepdims=True)
        cur = jnp.where(iota == idx, NEG, cur)
    Wk = jnp.where(Sm >= m, jnp.exp(Sm - m1), 0.0)
    denom = jnp.sum(Wk, axis=1, keepdims=True)
    A = Wk / denom
    per_img = lax.dot_general(A, Kseg, (((1,), (0,)), ((), ())),
                              preferred_element_type=jnp.float32)
    att = att_ref[0]
    att_flat = jnp.concatenate([att[r:r + 1, :] for r in range(R)], axis=1)
    biota = lax.broadcasted_iota(jnp.int32, (B, R * B), 0)
    colmod = lax.broadcasted_iota(jnp.int32, (B, R * B), 1) % B
    W3 = jnp.where(colmod == biota, att_flat, 0.0)
    ctx_img = lax.dot_general(W3, X, (((1,), (0,)), ((), ())),
                              preferred_element_type=jnp.float32)
    ctx_know = lax.dot_general(W3, per_img, (((1,), (0,)), ((), ())),
                               preferred_element_type=jnp.float32)
    q = q_ref[...]
    out_img_ref[...] = (0.5 * q + 0.5 * ctx_img)[None]
    out_know_ref[...] = (0.5 * q + 0.5 * ctx_know)[None]


def _t2_stage2(q, x_img, att, kstarts, kends, k_sorted):
    return pl.pallas_call(
        _t2_body,
        grid=(C,),
        in_specs=[
            pl.BlockSpec((B, D), lambda c: (0, 0)),
            pl.BlockSpec((1, R * B, D), lambda c: (c, 0, 0)),
            pl.BlockSpec((1, R, B), lambda c: (c, 0, 0)),
            pl.BlockSpec(memory_space=pltpu.MemorySpace.SMEM),
            pl.BlockSpec(memory_space=pltpu.MemorySpace.SMEM),
            pl.BlockSpec(memory_space=pltpu.MemorySpace.HBM),
        ],
        out_specs=[
            pl.BlockSpec((1, B, D), lambda c: (c, 0, 0)),
            pl.BlockSpec((1, B, D), lambda c: (c, 0, 0)),
        ],
        out_shape=[
            jax.ShapeDtypeStruct((C, B, D), jnp.float32),
            jax.ShapeDtypeStruct((C, B, D), jnp.float32),
        ],
        scratch_shapes=[
            pltpu.VMEM((KWIN, D), jnp.float32),
            pltpu.SemaphoreType.DMA,
        ],
    )(q, x_img, att, kstarts, kends, k_sorted)


def kernel(query_embeddings, all_image_embeddings, all_knowledge_embeddings,
           image_labels, knowledge_labels):
    classes = jnp.arange(C, dtype=image_labels.dtype)

    img_order = jnp.argsort(image_labels)
    sl = image_labels[img_order]
    starts = jnp.searchsorted(sl, classes, side='left').astype(jnp.int32)
    ends = jnp.searchsorted(sl, classes, side='right').astype(jnp.int32)
    img_order_p = jnp.concatenate([img_order.astype(jnp.int32),
                                   jnp.zeros((NPAD - N,), jnp.int32)])

    korder = jnp.argsort(knowledge_labels)
    kl = knowledge_labels[korder]
    kstarts = jnp.searchsorted(kl, classes, side='left').astype(jnp.int32)
    kends = jnp.searchsorted(kl, classes, side='right').astype(jnp.int32)
    korder_p = jnp.concatenate([korder.astype(jnp.int32),
                                jnp.zeros((KPAD - KPOOL,), jnp.int32)])

    k_sorted = _sc_gather(all_knowledge_embeddings, korder_p)
    q_pad = jnp.concatenate(
        [query_embeddings, jnp.zeros((B, D), jnp.float32)], axis=0)
    sims = _t0_sims(all_image_embeddings, q_pad)
    sims_sorted = _sc_gather(sims, img_order_p)
    att, pos = _t1_topk(starts, ends, sims_sorted)
    actual = img_order_p[pos.reshape(-1)]
    x_img = _sc_gather(all_image_embeddings, actual).reshape(C, R * B, D)
    out_img, out_know = _t2_stage2(query_embeddings, x_img, att,
                                   kstarts, kends, k_sorted)
    return (jnp.transpose(out_img, (1, 0, 2)),
            jnp.transpose(out_know, (1, 0, 2)))
